# Initial kernel scaffold; baseline (speedup 1.0000x reference)
import functools

import jax
import jax.numpy as jnp
from jax import lax
from jax.experimental import pallas as pl
from jax.experimental.pallas import tpu as pltpu
from jax.experimental.pallas import tpu_sc as plsc

N = 20000
IOU_T = 0.5
L = 16


def _make_nms(np_, nsub, interpret=False):
    C = np_ // nsub
    NV = C // L
    mesh = plsc.VectorSubcoreMesh(
        core_axis_name="c", subcore_axis_name="s", num_cores=1, num_subcores=nsub
    )

    def body(x1_hbm, y1_hbm, x2_hbm, y2_hbm, sc_hbm, out_hbm,
             cx1, cy1, cx2, cy2, car, cal, cio, surv, cnt_v, tmp,
             surv_sh, cnt_sh):
        w = lax.axis_index("s")
        base = w * C
        pltpu.sync_copy(x1_hbm.at[pl.ds(base, C)], cx1)
        pltpu.sync_copy(y1_hbm.at[pl.ds(base, C)], cy1)
        pltpu.sync_copy(x2_hbm.at[pl.ds(base, C)], cx2)
        pltpu.sync_copy(y2_hbm.at[pl.ds(base, C)], cy2)

        iota = lax.iota(jnp.int32, (L,))

        def init_v(v, _):
            sl = pl.ds(v * L, L)
            car[sl] = (cx2[sl] - cx1[sl]) * (cy2[sl] - cy1[sl])
            cal[sl] = jnp.full((L,), 1.0, jnp.float32)
            cio[sl] = iota + v * L
            return 0
        lax.fori_loop(0, NV, init_v, 0)

        def make_pass(masked):
            def vpass(bx1, by1, bx2, by2, ba, v0, p):
                def vbody(v, _):
                    sl = pl.ds(v * L, L)
                    xx1 = jnp.maximum(bx1, cx1[sl])
                    yy1 = jnp.maximum(by1, cy1[sl])
                    xx2 = jnp.minimum(bx2, cx2[sl])
                    yy2 = jnp.minimum(by2, cy2[sl])
                    iw = jnp.maximum(xx2 - xx1, 0.0)
                    ih = jnp.maximum(yy2 - yy1, 0.0)
                    inter = iw * ih
                    union = jnp.maximum(ba + car[sl] - inter, 1e-6)
                    over = inter > IOU_T * union
                    if masked:
                        over = over & (cio[sl] > p)
                    cal[sl] = jnp.where(over, 0.0, cal[sl])
                    return 0
                lax.fori_loop(v0, NV, vbody, 0)
            return vpass

        pass_masked = make_pass(True)
        pass_all = make_pass(False)

        def kblock(k, _):
            @pl.when(w == k)
            def _():
                def pbody(p, cnt):
                    def do(c):
                        bx1 = cx1[p]
                        by1 = cy1[p]
                        bx2 = cx2[p]
                        by2 = cy2[p]
                        ba = car[p]
                        v0 = lax.shift_right_logical(p, 4)
                        pass_masked(bx1, by1, bx2, by2, ba, v0, p)
                        surv[c] = bx1
                        surv[C + c] = by1
                        surv[2 * C + c] = bx2
                        surv[3 * C + c] = by2
                        surv[4 * C + c] = ba
                        return c + 1
                    return lax.cond(cal[p] > 0.5, do, lambda c: c, cnt)
                cnt = lax.fori_loop(0, C, pbody, 0)
                cnt_v[0] = cnt
                pltpu.sync_copy(surv, surv_sh)
                pltpu.sync_copy(cnt_v, cnt_sh)

            plsc.subcore_barrier()

            @pl.when(w > k)
            def _():
                pltpu.sync_copy(surv_sh, surv)
                pltpu.sync_copy(cnt_sh, cnt_v)
                def sbody(s, _):
                    bx1 = surv[s]
                    by1 = surv[C + s]
                    bx2 = surv[2 * C + s]
                    by2 = surv[3 * C + s]
                    ba = surv[4 * C + s]
                    pass_all(bx1, by1, bx2, by2, ba, 0, 0)
                    return 0
                lax.fori_loop(0, cnt_v[0], sbody, 0)

            plsc.subcore_barrier()
            return 0

        lax.fori_loop(0, nsub, kblock, 0)

        pltpu.sync_copy(sc_hbm.at[pl.ds(base, C)], tmp)

        def outv(v, _):
            sl = pl.ds(v * L, L)
            tmp[sl] = tmp[sl] * cal[sl]
            return 0
        lax.fori_loop(0, NV, outv, 0)
        pltpu.sync_copy(tmp, out_hbm.at[pl.ds(base, C)])

    return pl.kernel(
        body,
        out_type=jax.ShapeDtypeStruct((np_,), jnp.float32),
        mesh=mesh,
        scratch_types=[
            pltpu.VMEM((C,), jnp.float32),
            pltpu.VMEM((C,), jnp.float32),
            pltpu.VMEM((C,), jnp.float32),
            pltpu.VMEM((C,), jnp.float32),
            pltpu.VMEM((C,), jnp.float32),
            pltpu.VMEM((C,), jnp.float32),
            pltpu.VMEM((C,), jnp.int32),
            pltpu.VMEM((5 * C,), jnp.float32),
            pltpu.VMEM((16,), jnp.int32),
            pltpu.VMEM((C,), jnp.float32),
            pltpu.VMEM_SHARED((5 * C,), jnp.float32),
            pltpu.VMEM_SHARED((16,), jnp.int32),
        ],
        interpret=interpret,
    )


_NSUB = 16
_NP = 20480
_nms = _make_nms(_NP, _NSUB)


@jax.jit
def kernel(boxes, scores):
    order = jnp.argsort(-scores)
    boxes_s = jnp.take(boxes, order, axis=0)
    scores_s = jnp.take(scores, order, axis=0)
    pad = _NP - N
    x1 = jnp.pad(boxes_s[:, 0], (0, pad))
    y1 = jnp.pad(boxes_s[:, 1], (0, pad))
    x2 = jnp.pad(boxes_s[:, 2], (0, pad))
    y2 = jnp.pad(boxes_s[:, 3], (0, pad))
    sc = jnp.pad(scores_s, (0, pad))
    kept = _nms(x1, y1, x2, y2, sc)
    return kept[:N]


# speedup vs baseline: 6.0160x; 6.0160x over previous
import jax
import jax.numpy as jnp
from jax import lax
from jax.experimental import pallas as pl
from jax.experimental.pallas import tpu as pltpu
from jax.experimental.pallas import tpu_sc as plsc

N = 20000
IOU_T = 0.5
L = 16


def _make_nms(np_, nsub, interpret=False):
    C = np_ // nsub
    NV = C // L
    OAR = 4 * C
    OAL = 5 * C
    mesh = plsc.VectorSubcoreMesh(
        core_axis_name="c", subcore_axis_name="s", num_cores=1, num_subcores=nsub
    )

    def body(x1_hbm, y1_hbm, x2_hbm, y2_hbm, sc_hbm, out_hbm,
             me, ob, cio, tmp, blk_sh):
        w = lax.axis_index("s")
        base = w * C
        pltpu.sync_copy(x1_hbm.at[pl.ds(base, C)], me.at[pl.ds(0, C)])
        pltpu.sync_copy(y1_hbm.at[pl.ds(base, C)], me.at[pl.ds(C, C)])
        pltpu.sync_copy(x2_hbm.at[pl.ds(base, C)], me.at[pl.ds(2 * C, C)])
        pltpu.sync_copy(y2_hbm.at[pl.ds(base, C)], me.at[pl.ds(3 * C, C)])

        iota = lax.iota(jnp.int32, L)

        def init_v(v, _):
            o = v * L
            x1 = me[pl.ds(o, L)]
            y1 = me[pl.ds(C + o, L)]
            x2 = me[pl.ds(2 * C + o, L)]
            y2 = me[pl.ds(3 * C + o, L)]
            me[pl.ds(OAR + o, L)] = (x2 - x1) * (y2 - y1)
            me[pl.ds(OAL + o, L)] = jnp.full((L,), 1.0, jnp.float32)
            cio[pl.ds(o, L)] = iota + o
            return 0
        lax.fori_loop(0, NV, init_v, 0)

        def make_pass(masked):
            def vpass(bx1, by1, bx2, by2, ba, v0, p):
                def vbody(v, _):
                    o = v * L
                    xx1 = jnp.maximum(bx1, me[pl.ds(o, L)])
                    yy1 = jnp.maximum(by1, me[pl.ds(C + o, L)])
                    xx2 = jnp.minimum(bx2, me[pl.ds(2 * C + o, L)])
                    yy2 = jnp.minimum(by2, me[pl.ds(3 * C + o, L)])
                    iw = jnp.maximum(xx2 - xx1, 0.0)
                    ih = jnp.maximum(yy2 - yy1, 0.0)
                    inter = iw * ih
                    union = jnp.maximum(ba + me[pl.ds(OAR + o, L)] - inter, 1e-6)
                    over = inter > IOU_T * union
                    if masked:
                        over = over & (cio[pl.ds(o, L)] > p)
                    al = me[pl.ds(OAL + o, L)]
                    me[pl.ds(OAL + o, L)] = jnp.where(over, 0.0, al)
                    return 0
                lax.fori_loop(v0, NV, vbody, 0)
            return vpass

        pass_masked = make_pass(True)
        pass_all = make_pass(False)

        def kblock(k, _):
            @pl.when(w == k)
            def _():
                def gbody(g, _):
                    o = g * L
                    x1g = me[pl.ds(o, L)]
                    y1g = me[pl.ds(C + o, L)]
                    x2g = me[pl.ds(2 * C + o, L)]
                    y2g = me[pl.ds(3 * C + o, L)]
                    arg = me[pl.ds(OAR + o, L)]
                    for l in range(L):
                        alv = me[pl.ds(OAL + o, L)]
                        @pl.when(alv[l] > 0.5)
                        def _(l=l):
                            pass_masked(x1g[l], y1g[l], x2g[l], y2g[l],
                                        arg[l], g, o + l)
                    return 0
                lax.fori_loop(0, NV, gbody, 0)
                pltpu.sync_copy(me, blk_sh)

            plsc.subcore_barrier()

            @pl.when(w > k)
            def _():
                pltpu.sync_copy(blk_sh, ob)
                def sgroup(g, _):
                    o = g * L
                    x1g = ob[pl.ds(o, L)]
                    y1g = ob[pl.ds(C + o, L)]
                    x2g = ob[pl.ds(2 * C + o, L)]
                    y2g = ob[pl.ds(3 * C + o, L)]
                    arg = ob[pl.ds(OAR + o, L)]
                    alv = ob[pl.ds(OAL + o, L)]
                    for l in range(L):
                        @pl.when(alv[l] > 0.5)
                        def _(l=l):
                            pass_all(x1g[l], y1g[l], x2g[l], y2g[l],
                                     arg[l], 0, 0)
                    return 0
                lax.fori_loop(0, NV, sgroup, 0)

            plsc.subcore_barrier()
            return 0

        lax.fori_loop(0, nsub, kblock, 0)

        pltpu.sync_copy(sc_hbm.at[pl.ds(base, C)], tmp)

        def outv(v, _):
            o = v * L
            tmp[pl.ds(o, L)] = tmp[pl.ds(o, L)] * me[pl.ds(OAL + o, L)]
            return 0
        lax.fori_loop(0, NV, outv, 0)
        pltpu.sync_copy(tmp, out_hbm.at[pl.ds(base, C)])

    return pl.kernel(
        body,
        out_type=jax.ShapeDtypeStruct((np_,), jnp.float32),
        mesh=mesh,
        scratch_types=[
            pltpu.VMEM((6 * C,), jnp.float32),
            pltpu.VMEM((6 * C,), jnp.float32),
            pltpu.VMEM((C,), jnp.int32),
            pltpu.VMEM((C,), jnp.float32),
            pltpu.VMEM_SHARED((6 * C,), jnp.float32),
        ],
        interpret=interpret,
    )


_NSUB = 16
_NP = 20480
_nms = _make_nms(_NP, _NSUB)


@jax.jit
def kernel(boxes, scores):
    order = jnp.argsort(-scores)
    boxes_s = jnp.take(boxes, order, axis=0)
    scores_s = jnp.take(scores, order, axis=0)
    pad = _NP - N
    x1 = jnp.pad(boxes_s[:, 0], (0, pad))
    y1 = jnp.pad(boxes_s[:, 1], (0, pad))
    x2 = jnp.pad(boxes_s[:, 2], (0, pad))
    y2 = jnp.pad(boxes_s[:, 3], (0, pad))
    sc = jnp.pad(scores_s, (0, pad))
    kept = _nms(x1, y1, x2, y2, sc)
    return kept[:N]


# speedup vs baseline: 10.4861x; 1.7431x over previous
import jax
import jax.numpy as jnp
from jax import lax
from jax.experimental import pallas as pl
from jax.experimental.pallas import tpu as pltpu
from jax.experimental.pallas import tpu_sc as plsc

N = 20000
IOU_T = 0.5
L = 16
NSUB = 16
NP = 20480
B = NSUB * L
NB = NP // B
M = NP // B


def _make_nms(interpret=False):
    mesh = plsc.VectorSubcoreMesh(
        core_axis_name="c", subcore_axis_name="s", num_cores=1, num_subcores=NSUB
    )
    O1, O2, O3, O4 = NP, 2 * NP, 3 * NP, 4 * NP

    def body(x1_hbm, y1_hbm, x2_hbm, y2_hbm, sc_hbm, out_hbm,
             fx, al, flagwin, blkio, sco, flag_sh):
        w = lax.axis_index("s")
        pltpu.sync_copy(x1_hbm, fx.at[pl.ds(0, NP)])
        pltpu.sync_copy(y1_hbm, fx.at[pl.ds(O1, NP)])
        pltpu.sync_copy(x2_hbm, fx.at[pl.ds(O2, NP)])
        pltpu.sync_copy(y2_hbm, fx.at[pl.ds(O3, NP)])

        iota = lax.iota(jnp.int32, L)

        def init_v(v, _):
            o = v * L
            x1 = fx[pl.ds(o, L)]
            y1 = fx[pl.ds(O1 + o, L)]
            x2 = fx[pl.ds(O2 + o, L)]
            y2 = fx[pl.ds(O3 + o, L)]
            fx[pl.ds(O4 + o, L)] = (x2 - x1) * (y2 - y1)
            return 0
        lax.fori_loop(0, NP // L, init_v, 0)

        def init_al(v, _):
            al[pl.ds(v * L, L)] = jnp.full((L,), 1.0, jnp.float32)
            return 0
        lax.fori_loop(0, M, init_al, 0)

        def init_io(v, _):
            blkio[pl.ds(v * L, L)] = iota + v * L
            return 0
        lax.fori_loop(0, B // L, init_io, 0)

        def kblock(k, _):
            pltpu.sync_copy(al.at[pl.ds(k * L, L)], flag_sh.at[pl.ds(w * L, L)])
            plsc.subcore_barrier()
            pltpu.sync_copy(flag_sh, flagwin)
            plsc.subcore_barrier()

            kb = k * B

            def gbody(gp, _):
                o = kb + gp * L
                x1g = fx[pl.ds(o, L)]
                y1g = fx[pl.ds(O1 + o, L)]
                x2g = fx[pl.ds(O2 + o, L)]
                y2g = fx[pl.ds(O3 + o, L)]
                arg = fx[pl.ds(O4 + o, L)]
                for l in range(L):
                    alv = flagwin[pl.ds(gp * L, L)]
                    @pl.when(alv[l] > 0.5)
                    def _(l=l):
                        bx1 = x1g[l]
                        by1 = y1g[l]
                        bx2 = x2g[l]
                        by2 = y2g[l]
                        ba = arg[l]
                        p = gp * L + l

                        def vblk(v, _):
                            ob = kb + v * L
                            xx1 = jnp.maximum(bx1, fx[pl.ds(ob, L)])
                            yy1 = jnp.maximum(by1, fx[pl.ds(O1 + ob, L)])
                            xx2 = jnp.minimum(bx2, fx[pl.ds(O2 + ob, L)])
                            yy2 = jnp.minimum(by2, fx[pl.ds(O3 + ob, L)])
                            iw = jnp.maximum(xx2 - xx1, 0.0)
                            ih = jnp.maximum(yy2 - yy1, 0.0)
                            inter = iw * ih
                            union = jnp.maximum(ba + fx[pl.ds(O4 + ob, L)] - inter, 1e-6)
                            over = (inter > IOU_T * union) & (blkio[pl.ds(v * L, L)] > p)
                            fw = flagwin[pl.ds(v * L, L)]
                            flagwin[pl.ds(v * L, L)] = jnp.where(over, 0.0, fw)
                            return 0
                        lax.fori_loop(gp, B // L, vblk, 0)

                        def mbody(v2, _):
                            oc = v2 * B + w * L
                            xx1 = jnp.maximum(bx1, fx[pl.ds(oc, L)])
                            yy1 = jnp.maximum(by1, fx[pl.ds(O1 + oc, L)])
                            xx2 = jnp.minimum(bx2, fx[pl.ds(O2 + oc, L)])
                            yy2 = jnp.minimum(by2, fx[pl.ds(O3 + oc, L)])
                            iw = jnp.maximum(xx2 - xx1, 0.0)
                            ih = jnp.maximum(yy2 - yy1, 0.0)
                            inter = iw * ih
                            union = jnp.maximum(ba + fx[pl.ds(O4 + oc, L)] - inter, 1e-6)
                            over = inter > IOU_T * union
                            a = al[pl.ds(v2 * L, L)]
                            al[pl.ds(v2 * L, L)] = jnp.where(over, 0.0, a)
                            return 0
                        lax.fori_loop(k + 1, M, mbody, 0)
                return 0
            lax.fori_loop(0, B // L, gbody, 0)

            al[pl.ds(k * L, L)] = flagwin[pl.ds(w * L, L)]
            return 0

        lax.fori_loop(0, NB, kblock, 0)

        pltpu.sync_copy(sc_hbm.at[pl.ds(w * (NP // NSUB), NP // NSUB)], sco)

        def outv(v2, _):
            sco[pl.ds(v2 * L, L)] = sco[pl.ds(v2 * L, L)] * al[pl.ds(v2 * L, L)]
            return 0
        lax.fori_loop(0, M, outv, 0)
        pltpu.sync_copy(sco, out_hbm.at[pl.ds(w * (NP // NSUB), NP // NSUB)])

    return pl.kernel(
        body,
        out_type=jax.ShapeDtypeStruct((NP,), jnp.float32),
        mesh=mesh,
        scratch_types=[
            pltpu.VMEM((5 * NP,), jnp.float32),
            pltpu.VMEM((M * L,), jnp.float32),
            pltpu.VMEM((B,), jnp.float32),
            pltpu.VMEM((B,), jnp.int32),
            pltpu.VMEM((NP // NSUB,), jnp.float32),
            pltpu.VMEM_SHARED((B,), jnp.float32),
        ],
        interpret=interpret,
    )


_nms = _make_nms()


@jax.jit
def kernel(boxes, scores):
    order = jnp.argsort(-scores)
    boxes_s = jnp.take(boxes, order, axis=0)
    scores_s = jnp.take(scores, order, axis=0)
    pad = NP - N
    x1 = jnp.pad(boxes_s[:, 0], (0, pad))
    y1 = jnp.pad(boxes_s[:, 1], (0, pad))
    x2 = jnp.pad(boxes_s[:, 2], (0, pad))
    y2 = jnp.pad(boxes_s[:, 3], (0, pad))
    sc = jnp.pad(scores_s, (0, pad))
    scm = sc.reshape(NB, NSUB, L).transpose(1, 0, 2).reshape(NP)
    kept_own = _nms(x1, y1, x2, y2, scm)
    kept = kept_own.reshape(NSUB, NB, L).transpose(1, 0, 2).reshape(NP)
    return kept[:N]


# speedup vs baseline: 17.2682x; 1.6468x over previous
import jax
import jax.numpy as jnp
from jax import lax
from jax.experimental import pallas as pl
from jax.experimental.pallas import tpu as pltpu
from jax.experimental.pallas import tpu_sc as plsc

N = 20000
IOU_T = 0.5
L = 16
NSUB = 16
NP = 20480
B = NSUB * L
NB = NP // B
M = NP // B


def _make_nms(interpret=False):
    mesh = plsc.VectorSubcoreMesh(
        core_axis_name="c", subcore_axis_name="s", num_cores=1, num_subcores=NSUB
    )
    O1, O2, O3, O4 = NP, 2 * NP, 3 * NP, 4 * NP

    def body(x1_hbm, y1_hbm, x2_hbm, y2_hbm, sc_hbm, out_hbm,
             fx, al, flagwin, blkio, sco, flag_sh):
        w = lax.axis_index("s")
        pltpu.sync_copy(x1_hbm, fx.at[pl.ds(0, NP)])
        pltpu.sync_copy(y1_hbm, fx.at[pl.ds(O1, NP)])
        pltpu.sync_copy(x2_hbm, fx.at[pl.ds(O2, NP)])
        pltpu.sync_copy(y2_hbm, fx.at[pl.ds(O3, NP)])

        iota = lax.iota(jnp.int32, L)

        def init_v(v, _):
            o = v * L
            x1 = fx[pl.ds(o, L)]
            y1 = fx[pl.ds(O1 + o, L)]
            x2 = fx[pl.ds(O2 + o, L)]
            y2 = fx[pl.ds(O3 + o, L)]
            fx[pl.ds(O4 + o, L)] = (x2 - x1) * (y2 - y1)
            return 0
        lax.fori_loop(0, NP // L, init_v, 0)

        def init_al(v, _):
            al[pl.ds(v * L, L)] = jnp.full((L,), 1.0, jnp.float32)
            return 0
        lax.fori_loop(0, M, init_al, 0)

        def init_io(v, _):
            blkio[pl.ds(v * L, L)] = iota + v * L
            return 0
        lax.fori_loop(0, B // L, init_io, 0)

        def kblock(k, _):
            pltpu.sync_copy(al.at[pl.ds(k * L, L)], flag_sh.at[pl.ds(w * L, L)])
            plsc.subcore_barrier()
            pltpu.sync_copy(flag_sh, flagwin)
            plsc.subcore_barrier()

            kb = k * B

            def gbody(gp, _):
                o = kb + gp * L
                x1g = fx[pl.ds(o, L)]
                y1g = fx[pl.ds(O1 + o, L)]
                x2g = fx[pl.ds(O2 + o, L)]
                y2g = fx[pl.ds(O3 + o, L)]
                arg = fx[pl.ds(O4 + o, L)]
                for l in range(L):
                    alv = flagwin[pl.ds(gp * L, L)]
                    @pl.when(alv[l] > 0.5)
                    def _(l=l):
                        bx1 = x1g[l]
                        by1 = y1g[l]
                        bx2 = x2g[l]
                        by2 = y2g[l]
                        ba = arg[l]
                        p = gp * L + l

                        @plsc.parallel_loop(gp, B // L, unroll=2)
                        def vblk(v):
                            ob = kb + v * L
                            xx1 = jnp.maximum(bx1, fx[pl.ds(ob, L)])
                            yy1 = jnp.maximum(by1, fx[pl.ds(O1 + ob, L)])
                            xx2 = jnp.minimum(bx2, fx[pl.ds(O2 + ob, L)])
                            yy2 = jnp.minimum(by2, fx[pl.ds(O3 + ob, L)])
                            iw = jnp.maximum(xx2 - xx1, 0.0)
                            ih = jnp.maximum(yy2 - yy1, 0.0)
                            inter = iw * ih
                            union = jnp.maximum(ba + fx[pl.ds(O4 + ob, L)] - inter, 1e-6)
                            over = (inter > IOU_T * union) & (blkio[pl.ds(v * L, L)] > p)
                            fw = flagwin[pl.ds(v * L, L)]
                            flagwin[pl.ds(v * L, L)] = jnp.where(over, 0.0, fw)

                        @plsc.parallel_loop(k + 1, M, unroll=4)
                        def mbody(v2):
                            oc = v2 * B + w * L
                            xx1 = jnp.maximum(bx1, fx[pl.ds(oc, L)])
                            yy1 = jnp.maximum(by1, fx[pl.ds(O1 + oc, L)])
                            xx2 = jnp.minimum(bx2, fx[pl.ds(O2 + oc, L)])
                            yy2 = jnp.minimum(by2, fx[pl.ds(O3 + oc, L)])
                            iw = jnp.maximum(xx2 - xx1, 0.0)
                            ih = jnp.maximum(yy2 - yy1, 0.0)
                            inter = iw * ih
                            union = jnp.maximum(ba + fx[pl.ds(O4 + oc, L)] - inter, 1e-6)
                            over = inter > IOU_T * union
                            a = al[pl.ds(v2 * L, L)]
                            al[pl.ds(v2 * L, L)] = jnp.where(over, 0.0, a)
                return 0
            lax.fori_loop(0, B // L, gbody, 0)

            al[pl.ds(k * L, L)] = flagwin[pl.ds(w * L, L)]
            return 0

        lax.fori_loop(0, NB, kblock, 0)

        pltpu.sync_copy(sc_hbm.at[pl.ds(w * (NP // NSUB), NP // NSUB)], sco)

        def outv(v2, _):
            sco[pl.ds(v2 * L, L)] = sco[pl.ds(v2 * L, L)] * al[pl.ds(v2 * L, L)]
            return 0
        lax.fori_loop(0, M, outv, 0)
        pltpu.sync_copy(sco, out_hbm.at[pl.ds(w * (NP // NSUB), NP // NSUB)])

    return pl.kernel(
        body,
        out_type=jax.ShapeDtypeStruct((NP,), jnp.float32),
        mesh=mesh,
        scratch_types=[
            pltpu.VMEM((5 * NP,), jnp.float32),
            pltpu.VMEM((M * L,), jnp.float32),
            pltpu.VMEM((B,), jnp.float32),
            pltpu.VMEM((B,), jnp.int32),
            pltpu.VMEM((NP // NSUB,), jnp.float32),
            pltpu.VMEM_SHARED((B,), jnp.float32),
        ],
        interpret=interpret,
    )


_nms = _make_nms()


@jax.jit
def kernel(boxes, scores):
    order = jnp.argsort(-scores)
    boxes_s = jnp.take(boxes, order, axis=0)
    scores_s = jnp.take(scores, order, axis=0)
    pad = NP - N
    x1 = jnp.pad(boxes_s[:, 0], (0, pad))
    y1 = jnp.pad(boxes_s[:, 1], (0, pad))
    x2 = jnp.pad(boxes_s[:, 2], (0, pad))
    y2 = jnp.pad(boxes_s[:, 3], (0, pad))
    sc = jnp.pad(scores_s, (0, pad))
    scm = sc.reshape(NB, NSUB, L).transpose(1, 0, 2).reshape(NP)
    kept_own = _nms(x1, y1, x2, y2, scm)
    kept = kept_own.reshape(NSUB, NB, L).transpose(1, 0, 2).reshape(NP)
    return kept[:N]
